# Initial kernel scaffold; baseline (speedup 1.0000x reference)
import jax
import jax.numpy as jnp
from jax.experimental import pallas as pl


def kernel(embed_idx, ohes, tables, W, b):
    raise NotImplementedError("write your pallas kernel here")


# speedup vs baseline: 1.8998x; 1.8998x over previous
import functools

import jax
import jax.numpy as jnp
from jax import lax
from jax.experimental import pallas as pl
from jax.experimental.pallas import tpu as pltpu
from jax.experimental.pallas import tpu_sc as plsc

N_FIELDS = 26
VOCAB = 100000
EMB = 16
OHE = 100
HID = 128
BATCH = 16384
EMB_FEAT = N_FIELDS * EMB
TOTAL_ROWS = BATCH * N_FIELDS

_NC = 2
_NS = 16
_NW = _NC * _NS
_PER_W = TOTAL_ROWS // _NW
_CHUNK = 3328
_N_CHUNKS = _PER_W // _CHUNK


def _gather_body(table_hbm, idx_hbm, out_hbm, idx_v, rows_v, sem):
    wid = lax.axis_index("s") * _NC + lax.axis_index("c")
    base = wid * _PER_W
    for c in range(_N_CHUNKS):
        off = base + c * _CHUNK
        pltpu.sync_copy(idx_hbm.at[pl.ds(off, _CHUNK)], idx_v)
        pltpu.async_copy(table_hbm.at[idx_v], rows_v, sem).wait()
        pltpu.sync_copy(rows_v, out_hbm.at[pl.ds(off, _CHUNK)])


_gather = functools.partial(
    pl.kernel,
    mesh=plsc.VectorSubcoreMesh(core_axis_name="c", subcore_axis_name="s"),
    out_type=jax.ShapeDtypeStruct((TOTAL_ROWS, EMB), jnp.float32),
    scratch_types=[
        pltpu.VMEM((_CHUNK,), jnp.int32),
        pltpu.VMEM((_CHUNK, EMB), jnp.float32),
        pltpu.SemaphoreType.DMA,
    ],
    compiler_params=pltpu.CompilerParams(use_tc_tiling_on_sc=False),
)(_gather_body)


_BM = 2048


def _mm_body(e_ref, o_ref, w1_ref, w2_ref, b_ref, out_ref):
    acc = jnp.dot(e_ref[...], w1_ref[...], preferred_element_type=jnp.float32)
    acc = acc + jnp.dot(o_ref[...], w2_ref[...], preferred_element_type=jnp.float32)
    out_ref[...] = acc + b_ref[...]


_mm = pl.pallas_call(
    _mm_body,
    grid=(BATCH // _BM,),
    in_specs=[
        pl.BlockSpec((_BM, EMB_FEAT), lambda i: (i, 0)),
        pl.BlockSpec((_BM, OHE), lambda i: (i, 0)),
        pl.BlockSpec((EMB_FEAT, HID), lambda i: (0, 0)),
        pl.BlockSpec((OHE, HID), lambda i: (0, 0)),
        pl.BlockSpec((1, HID), lambda i: (0, 0)),
    ],
    out_specs=pl.BlockSpec((_BM, HID), lambda i: (i, 0)),
    out_shape=jax.ShapeDtypeStruct((BATCH, HID), jnp.float32),
)


@jax.jit
def kernel(embed_idx, ohes, tables, W, b):
    flat_tables = tables.reshape(N_FIELDS * VOCAB, EMB)
    offs = jnp.arange(N_FIELDS, dtype=jnp.int32) * VOCAB
    flat_idx = (embed_idx.astype(jnp.int32) + offs[None, :]).reshape(-1)
    e = _gather(flat_tables, flat_idx)
    e = e.reshape(BATCH, EMB_FEAT)
    return _mm(e, ohes, W[:EMB_FEAT], W[EMB_FEAT:], b.reshape(1, HID))
